# baseline (speedup 1.0000x reference)
import jax
import jax.numpy as jnp
from jax import lax
from jax.experimental import pallas as pl
from jax.experimental.pallas import tpu as pltpu
from jax.experimental.pallas import tpu_sc as plsc

S = 2048
D = 1024
E = 8
F = 512
T = 128
NBLK = S * 2 // T + E - 1
NROWS = NBLK * T
NW = 32
TOK_W = S // NW
DCH = 32
CCH = 16
NCC = TOK_W // CCH


def _router_body(x_ref, wg_ref, d0_ref, d1_ref, p0_ref, p1_ref, bemap_ref):
    x = x_ref[...]
    scores = jnp.dot(x, wg_ref[...], preferred_element_type=jnp.float32)
    iota_e = lax.broadcasted_iota(jnp.int32, (S, E), 1)

    m1 = jnp.max(scores, axis=-1, keepdims=True)
    idx1 = jnp.min(jnp.where(scores == m1, iota_e, E), axis=-1, keepdims=True)
    oh1 = iota_e == idx1
    scores2 = jnp.where(oh1, -jnp.inf, scores)
    m2 = jnp.max(scores2, axis=-1, keepdims=True)
    idx2 = jnp.min(jnp.where(scores2 == m2, iota_e, E), axis=-1, keepdims=True)
    oh2 = iota_e == idx2
    t = jnp.exp(m2 - m1)
    p0 = 1.0 / (1.0 + t)
    p1 = t / (1.0 + t)

    oh1f = oh1.astype(jnp.float32)
    oh2f = oh2.astype(jnp.float32)
    oh = jnp.concatenate([oh1f, oh2f], axis=1)
    ir = lax.broadcasted_iota(jnp.int32, (S, S), 0)
    ic = lax.broadcasted_iota(jnp.int32, (S, S), 1)
    tril = (ir >= ic).astype(jnp.float32)
    csum = jnp.dot(tril, oh, preferred_element_type=jnp.float32)
    c1 = csum[:, :E]
    c2 = csum[:, E:]

    t1 = jnp.sum(oh1f, axis=0, keepdims=True)
    t2 = jnp.sum(oh2f, axis=0, keepdims=True)
    cnt = t1 + t2
    pc = jnp.floor((cnt + (T - 1)) * (1.0 / T)) * T
    iu_r = lax.broadcasted_iota(jnp.int32, (E, E), 0)
    iu_c = lax.broadcasted_iota(jnp.int32, (E, E), 1)
    upper = (iu_r < iu_c).astype(jnp.float32)
    off = jnp.dot(pc, upper, preferred_element_type=jnp.float32)

    d0 = jnp.sum(oh1f * (off + c1 - 1.0), axis=1, keepdims=True)
    d1 = jnp.sum(oh2f * (off + t1 + c2 - 1.0), axis=1, keepdims=True)
    d0_ref[...] = d0.astype(jnp.int32)
    d1_ref[...] = d1.astype(jnp.int32)
    p0_ref[...] = jnp.broadcast_to(p0, (S, 16))
    p1_ref[...] = jnp.broadcast_to(p1, (S, 16))

    offend = off + pc
    ib = lax.broadcasted_iota(jnp.int32, (48, E), 0).astype(jnp.float32) * T
    be = jnp.sum((offend <= ib).astype(jnp.int32), axis=1, keepdims=True)
    be = jnp.minimum(be, E - 1)
    nact = (jnp.sum(pc, axis=1, keepdims=True) * (1.0 / T)).astype(jnp.int32)
    rowi = lax.broadcasted_iota(jnp.int32, (48, 1), 0)
    bevals = jnp.where(rowi == NBLK, nact, be)
    bemap_ref[...] = jnp.broadcast_to(bevals, (48, 128))


def _router(xs, Wg):
    return pl.pallas_call(
        _router_body,
        in_specs=[
            pl.BlockSpec((S, D), lambda: (0, 0)),
            pl.BlockSpec((D, E), lambda: (0, 0)),
        ],
        out_specs=[
            pl.BlockSpec((S, 1), lambda: (0, 0)),
            pl.BlockSpec((S, 1), lambda: (0, 0)),
            pl.BlockSpec((S, 16), lambda: (0, 0)),
            pl.BlockSpec((S, 16), lambda: (0, 0)),
            pl.BlockSpec((48, 128), lambda: (0, 0)),
        ],
        out_shape=[
            jax.ShapeDtypeStruct((S, 1), jnp.int32),
            jax.ShapeDtypeStruct((S, 1), jnp.int32),
            jax.ShapeDtypeStruct((S, 16), jnp.float32),
            jax.ShapeDtypeStruct((S, 16), jnp.float32),
            jax.ShapeDtypeStruct((48, 128), jnp.int32),
        ],
    )(xs, Wg)


def _dispatch_body(x_hbm, d0_hbm, d1_hbm, xs_hbm,
                   idx0a_v, idx1a_v, idx0b_v, idx1b_v, rowsa_v, rowsb_v,
                   seml, sems):
    wid = lax.axis_index("s") * 2 + lax.axis_index("c")
    base = wid * TOK_W
    tb0 = base
    tb1 = base + DCH
    pltpu.sync_copy(d0_hbm.at[pl.ds(tb0, DCH)], idx0a_v)
    pltpu.sync_copy(d1_hbm.at[pl.ds(tb0, DCH)], idx1a_v)
    pltpu.sync_copy(d0_hbm.at[pl.ds(tb1, DCH)], idx0b_v)
    pltpu.sync_copy(d1_hbm.at[pl.ds(tb1, DCH)], idx1b_v)
    la = pltpu.async_copy(x_hbm.at[pl.ds(tb0, DCH)], rowsa_v, seml)
    lb = pltpu.async_copy(x_hbm.at[pl.ds(tb1, DCH)], rowsb_v, seml)
    la.wait()
    s0 = pltpu.async_copy(rowsa_v, xs_hbm.at[idx0a_v], sems)
    s1 = pltpu.async_copy(rowsa_v, xs_hbm.at[idx1a_v], sems)
    lb.wait()
    s2 = pltpu.async_copy(rowsb_v, xs_hbm.at[idx0b_v], sems)
    s3 = pltpu.async_copy(rowsb_v, xs_hbm.at[idx1b_v], sems)
    s0.wait()
    s1.wait()
    s2.wait()
    s3.wait()


def _dispatch(xs, d0, d1):
    mesh = plsc.VectorSubcoreMesh(core_axis_name="c", subcore_axis_name="s")
    run = pl.kernel(
        _dispatch_body,
        out_type=jax.ShapeDtypeStruct((NROWS, D), jnp.float32),
        mesh=mesh,
        scratch_types=[
            pltpu.VMEM((DCH,), jnp.int32),
            pltpu.VMEM((DCH,), jnp.int32),
            pltpu.VMEM((DCH,), jnp.int32),
            pltpu.VMEM((DCH,), jnp.int32),
            pltpu.VMEM((DCH, D), jnp.float32),
            pltpu.VMEM((DCH, D), jnp.float32),
            pltpu.SemaphoreType.DMA,
            pltpu.SemaphoreType.DMA,
        ],
    )
    return run(xs, d0, d1)


def _gmm_body(be_ref, xs_ref, w1_ref, w2_ref, w3_ref, ys_ref):
    i = pl.program_id(0)
    nact = be_ref[NBLK, 0]

    @pl.when(i < nact)
    def _():
        xb = xs_ref[...]
        a = jnp.dot(xb, w1_ref[0], preferred_element_type=jnp.float32)
        b = jnp.dot(xb, w2_ref[0], preferred_element_type=jnp.float32)
        h = (a * lax.logistic(a)) * b
        ys_ref[...] = jnp.dot(h, w3_ref[0], preferred_element_type=jnp.float32)


def _gmm(be_arr, xs_sorted, W1, W2, W3):
    grid_spec = pltpu.PrefetchScalarGridSpec(
        num_scalar_prefetch=1,
        grid=(NBLK,),
        in_specs=[
            pl.BlockSpec((T, D), lambda i, be: (i, 0)),
            pl.BlockSpec((1, D, F), lambda i, be: (be[i, 0], 0, 0)),
            pl.BlockSpec((1, D, F), lambda i, be: (be[i, 0], 0, 0)),
            pl.BlockSpec((1, F, D), lambda i, be: (be[i, 0], 0, 0)),
        ],
        out_specs=pl.BlockSpec((T, D), lambda i, be: (i, 0)),
    )
    return pl.pallas_call(
        _gmm_body,
        grid_spec=grid_spec,
        out_shape=jax.ShapeDtypeStruct((NROWS, D), jnp.float32),
    )(be_arr, xs_sorted, W1, W2, W3)


def _combine_body(ys_hbm, d0_hbm, d1_hbm, p0_hbm, p1_hbm, out_hbm,
                  idx0_v, idx1_v, prb0_v, prb1_v,
                  r0a_v, r1a_v, r0b_v, r1b_v, sga, sgb, swa, swb):
    wid = lax.axis_index("s") * 2 + lax.axis_index("c")
    base = wid * TOK_W
    pltpu.sync_copy(p0_hbm.at[pl.ds(base, TOK_W)], prb0_v)
    pltpu.sync_copy(p1_hbm.at[pl.ds(base, TOK_W)], prb1_v)

    rbufs = ((r0a_v, r1a_v, sga, swa), (r0b_v, r1b_v, sgb, swb))
    idxs = (idx0_v, idx1_v)

    def fire(c):
        r0, r1, sg, _ = rbufs[c % 2]
        i0 = idxs[0].at[pl.ds(c * CCH, CCH)]
        i1 = idxs[1].at[pl.ds(c * CCH, CCH)]
        g0 = pltpu.async_copy(ys_hbm.at[i0], r0, sg)
        g1 = pltpu.async_copy(ys_hbm.at[i1], r1, sg)
        return g0, g1

    pltpu.sync_copy(d0_hbm.at[pl.ds(base, TOK_W)], idx0_v)
    pltpu.sync_copy(d1_hbm.at[pl.ds(base, TOK_W)], idx1_v)

    pending = [fire(0), fire(1)]
    writes = [None, None]
    for c in range(NCC):
        r0, r1, sg, sw = rbufs[c % 2]
        g0, g1 = pending[c % 2]
        g0.wait()
        g1.wait()

        def row_loop(i, _):
            pv0 = prb0_v[c * CCH + i]
            pv1 = prb1_v[c * CCH + i]
            for j in range(D // 16):
                sl = pl.ds(j * 16, 16)
                r0[i, sl] = pv0 * r0[i, sl] + pv1 * r1[i, sl]
            return 0

        lax.fori_loop(0, CCH, row_loop, 0)
        w = pltpu.async_copy(r0, out_hbm.at[pl.ds(base + c * CCH, CCH)], sw)
        writes[c % 2] = w
        if c + 2 < NCC:
            writes[c % 2].wait()
            pending[c % 2] = fire(c + 2)
    for w in writes:
        if w is not None:
            w.wait()


def _combine(ys_sorted, d0, d1, p0rep, p1rep):
    mesh = plsc.VectorSubcoreMesh(core_axis_name="c", subcore_axis_name="s")
    run = pl.kernel(
        _combine_body,
        out_type=jax.ShapeDtypeStruct((S, D), jnp.float32),
        mesh=mesh,
        scratch_types=[
            pltpu.VMEM((TOK_W,), jnp.int32),
            pltpu.VMEM((TOK_W,), jnp.int32),
            pltpu.VMEM((TOK_W, 16), jnp.float32),
            pltpu.VMEM((TOK_W, 16), jnp.float32),
            pltpu.VMEM((CCH, D), jnp.float32),
            pltpu.VMEM((CCH, D), jnp.float32),
            pltpu.VMEM((CCH, D), jnp.float32),
            pltpu.VMEM((CCH, D), jnp.float32),
            pltpu.SemaphoreType.DMA,
            pltpu.SemaphoreType.DMA,
            pltpu.SemaphoreType.DMA,
            pltpu.SemaphoreType.DMA,
        ],
    )
    return run(ys_sorted, d0, d1, p0rep, p1rep)


def kernel(x, Wg, W1, W2, W3):
    B = x.shape[0]
    xs = x.reshape(S, D)

    d0c, d1c, p0rep, p1rep, bemap = _router(xs, Wg)
    d0 = d0c.reshape(S)
    d1 = d1c.reshape(S)
    xs_sorted = _dispatch(xs, d0, d1)
    ys_sorted = _gmm(bemap, xs_sorted, W1, W2, W3)
    out = _combine(ys_sorted, d0, d1, p0rep, p1rep)
    return out.reshape(B, S, D)


# speedup vs baseline: 1.2510x; 1.2510x over previous
import jax
import jax.numpy as jnp
from jax import lax
from jax.experimental import pallas as pl

S, D, E, F = 2048, 1024, 8, 512
NS = 4
SB = S // NS


def _moe_dense_kernel(x_ref, wg_ref, w1_ref, w2_ref, w3_ref, out_ref):
    e = pl.program_id(0)
    s = pl.program_id(1)
    xs = x_ref[pl.ds(s * SB, SB), :]

    scores = jnp.dot(xs, wg_ref[...], preferred_element_type=jnp.float32)
    iota = lax.broadcasted_iota(jnp.int32, scores.shape, 1)
    m1 = jnp.max(scores, axis=-1, keepdims=True)
    idx1 = jnp.min(jnp.where(scores == m1, iota, E), axis=-1, keepdims=True)
    oh1 = iota == idx1
    scores2 = jnp.where(oh1, -jnp.inf, scores)
    m2 = jnp.max(scores2, axis=-1, keepdims=True)
    idx2 = jnp.min(jnp.where(scores2 == m2, iota, E), axis=-1, keepdims=True)
    oh2 = iota == idx2
    t = jnp.exp(m2 - m1)
    p1 = 1.0 / (1.0 + t)
    p2 = t / (1.0 + t)
    gates = p1 * oh1.astype(jnp.float32) + p2 * oh2.astype(jnp.float32)
    gate_e = jnp.sum(jnp.where(iota == e, gates, 0.0), axis=-1, keepdims=True)

    a = jnp.dot(xs, w1_ref[0], preferred_element_type=jnp.float32)
    b = jnp.dot(xs, w2_ref[0], preferred_element_type=jnp.float32)
    h = (a * lax.logistic(a)) * b
    y = jnp.dot(h, w3_ref[0], preferred_element_type=jnp.float32)

    @pl.when(e == 0)
    def _():
        out_ref[pl.ds(s * SB, SB), :] = gate_e * y

    @pl.when(e > 0)
    def _():
        out_ref[pl.ds(s * SB, SB), :] += gate_e * y


def kernel(x, Wg, W1, W2, W3):
    B = x.shape[0]
    xs = x.reshape(S, D)

    out = pl.pallas_call(
        _moe_dense_kernel,
        grid=(E, NS),
        in_specs=[
            pl.BlockSpec((S, D), lambda e, s: (0, 0)),
            pl.BlockSpec((D, E), lambda e, s: (0, 0)),
            pl.BlockSpec((1, D, F), lambda e, s: (e, 0, 0)),
            pl.BlockSpec((1, D, F), lambda e, s: (e, 0, 0)),
            pl.BlockSpec((1, F, D), lambda e, s: (e, 0, 0)),
        ],
        out_specs=pl.BlockSpec((S, D), lambda e, s: (0, 0)),
        out_shape=jax.ShapeDtypeStruct((S, D), jnp.float32),
    )(xs, Wg, W1, W2, W3)
    return out.reshape(B, S, D)


# speedup vs baseline: 1.4895x; 1.1906x over previous
import jax
import jax.numpy as jnp
from jax import lax
from jax.experimental import pallas as pl
from jax.experimental.pallas import tpu as pltpu

S, D, E, F = 2048, 1024, 8, 512


def _moe_dense_kernel(x_ref, wg_ref, w1_ref, w2_ref, w3_ref, out_ref, g_ref):
    e = pl.program_id(0)
    xs = x_ref[...]

    @pl.when(e == 0)
    def _():
        scores = jnp.dot(xs, wg_ref[...], preferred_element_type=jnp.float32)
        iota = lax.broadcasted_iota(jnp.int32, scores.shape, 1)
        m1 = jnp.max(scores, axis=-1, keepdims=True)
        idx1 = jnp.min(jnp.where(scores == m1, iota, E), axis=-1, keepdims=True)
        oh1 = iota == idx1
        scores2 = jnp.where(oh1, -jnp.inf, scores)
        m2 = jnp.max(scores2, axis=-1, keepdims=True)
        idx2 = jnp.min(jnp.where(scores2 == m2, iota, E), axis=-1, keepdims=True)
        oh2 = iota == idx2
        t = jnp.exp(m2 - m1)
        p1 = 1.0 / (1.0 + t)
        p2 = t / (1.0 + t)
        g_ref[...] = p1 * oh1.astype(jnp.float32) + p2 * oh2.astype(jnp.float32)

    iota = lax.broadcasted_iota(jnp.int32, (S, E), 1)
    gate_e = jnp.sum(jnp.where(iota == e, g_ref[...], 0.0), axis=-1, keepdims=True)

    a = jnp.dot(xs, w1_ref[0], preferred_element_type=jnp.float32)
    b = jnp.dot(xs, w2_ref[0], preferred_element_type=jnp.float32)
    h = (a * lax.logistic(a)) * b
    y = jnp.dot(h, w3_ref[0], preferred_element_type=jnp.float32)

    @pl.when(e == 0)
    def _():
        out_ref[...] = gate_e * y

    @pl.when(e > 0)
    def _():
        out_ref[...] += gate_e * y


def kernel(x, Wg, W1, W2, W3):
    B = x.shape[0]
    xs = x.reshape(S, D)

    out = pl.pallas_call(
        _moe_dense_kernel,
        grid=(E,),
        in_specs=[
            pl.BlockSpec((S, D), lambda e: (0, 0)),
            pl.BlockSpec((D, E), lambda e: (0, 0)),
            pl.BlockSpec((1, D, F), lambda e: (e, 0, 0)),
            pl.BlockSpec((1, D, F), lambda e: (e, 0, 0)),
            pl.BlockSpec((1, F, D), lambda e: (e, 0, 0)),
        ],
        out_specs=pl.BlockSpec((S, D), lambda e: (0, 0)),
        out_shape=jax.ShapeDtypeStruct((S, D), jnp.float32),
        scratch_shapes=[pltpu.VMEM((S, E), jnp.float32)],
    )(xs, Wg, W1, W2, W3)
    return out.reshape(B, S, D)
